# baseline (speedup 1.0000x reference)
import functools

import jax
import jax.numpy as jnp
from jax import lax
from jax.experimental import pallas as pl
from jax.experimental.pallas import tpu as pltpu
from jax.experimental.pallas import tpu_sc as plsc

B = 16384
C = 1000
MARGIN = 1.0

BSC = 4096
BT = B - BSC

BCOL = 4096
GRID = BT // BCOL

NC = 2
NS = 16
L = 16
NW = NC * NS
SW = BSC // NW
NG = SW // L
UNROLL = 8


def _tc_hinge_body(xt_ref, y_ref, out_ref):
    pi = pl.program_id(0)
    xt = xt_ref[...]
    yv = y_ref[0, 0, :]
    yrow = yv.reshape(1, BCOL)
    cls = lax.broadcasted_iota(jnp.int32, (C, BCOL), 0)
    oy = jnp.sum(jnp.where(cls == yrow, xt, 0.0), axis=0, keepdims=True)
    s = jnp.sum(jnp.maximum(xt - oy + MARGIN, 0.0))

    @pl.when(pi == 0)
    def _init():
        out_ref[0, 0] = 0.0

    out_ref[0, 0] += s


_tc_hinge = pl.pallas_call(
    _tc_hinge_body,
    grid=(GRID,),
    in_specs=[
        pl.BlockSpec((C, BCOL), lambda i: (0, i)),
        pl.BlockSpec((1, 1, BCOL), lambda i: (i, 0, 0)),
    ],
    out_specs=pl.BlockSpec((1, 1), lambda i: (0, 0), memory_space=pltpu.SMEM),
    out_shape=jax.ShapeDtypeStruct((1, 1), jnp.float32),
)


def _sc_hinge_body(xt_hbm, y_hbm, out_hbm, ybuf, xbuf, accbuf, sem):
    wid = lax.axis_index("s") * NC + lax.axis_index("c")
    soff = pl.multiple_of(BT + wid * SW, SW)
    pltpu.sync_copy(y_hbm.at[pl.ds(soff, SW)], ybuf)
    pltpu.async_copy(xt_hbm.at[:, pl.ds(soff, SW)], xbuf, sem).wait()

    NA = 4

    accs = tuple(jnp.zeros((L,), jnp.float32) for _ in range(NA))
    for q in range(NG):
        yv = ybuf[pl.ds(q * L, L)]

        def s1(i, oyvs, yv=yv, q=q):
            c0 = i * UNROLL
            d = yv - c0
            oyvs = list(oyvs)
            for k in range(UNROLL):
                v = xbuf[c0 + k, pl.ds(q * L, L)]
                oyvs[k % NA] = jnp.where(d == k, v, oyvs[k % NA])
            return tuple(oyvs)

        oyvs = lax.fori_loop(0, C // UNROLL, s1,
                             tuple(jnp.zeros((L,), jnp.float32)
                                   for _ in range(NA)))
        ym = (oyvs[0] + oyvs[1]) + (oyvs[2] + oyvs[3]) - MARGIN

        def s2(i, accs_, ym=ym, q=q):
            c0 = i * UNROLL
            accs_ = list(accs_)
            for k in range(UNROLL):
                v = xbuf[c0 + k, pl.ds(q * L, L)]
                accs_[k % NA] = accs_[k % NA] + jnp.maximum(v - ym, 0.0)
            return tuple(accs_)

        accs = lax.fori_loop(0, C // UNROLL, s2, accs)

    accbuf[...] = (accs[0] + accs[1]) + (accs[2] + accs[3])
    pltpu.sync_copy(accbuf, out_hbm.at[pl.ds(wid * L, L)])


@functools.cache
def _sc_hinge():
    return pl.kernel(
        _sc_hinge_body,
        out_type=jax.ShapeDtypeStruct((NW * L,), jnp.float32),
        mesh=plsc.VectorSubcoreMesh(core_axis_name="c", subcore_axis_name="s",
                                    num_cores=NC, num_subcores=NS),
        scratch_types=[
            pltpu.VMEM((SW,), jnp.int32),
            pltpu.VMEM((C, SW), jnp.float32),
            pltpu.VMEM((L,), jnp.float32),
            pltpu.SemaphoreType.DMA,
        ],
    )


def kernel(output, y):
    y32 = y.astype(jnp.int32)
    xt = output.T
    sc_partials = _sc_hinge()(xt, y32)
    y3 = y32[:BT].reshape(GRID, 1, BCOL)
    tc_partial = _tc_hinge(xt, y3)
    total = tc_partial[0, 0] + jnp.sum(sc_partials)
    return (total - float(B)) / float(B)


# speedup vs baseline: 1.0475x; 1.0475x over previous
import functools

import jax
import jax.numpy as jnp
from jax import lax
from jax.experimental import pallas as pl
from jax.experimental.pallas import tpu as pltpu
from jax.experimental.pallas import tpu_sc as plsc

B = 16384
C = 1000
MARGIN = 1.0

BSC = 4096
BT = B - BSC

BCOL = 2048
GRID = BT // BCOL

NC = 2
NS = 16
L = 16
NW = NC * NS
SW = BSC // NW
NG = SW // L
UNROLL = 8


def _tc_hinge_body(xt_ref, y_ref, out_ref):
    pi = pl.program_id(0)
    xt = xt_ref[...]
    yv = y_ref[0, 0, :]
    yrow = yv.reshape(1, BCOL)
    cls = lax.broadcasted_iota(jnp.int32, (C, BCOL), 0)
    oy = jnp.sum(jnp.where(cls == yrow, xt, 0.0), axis=0, keepdims=True)
    s = jnp.sum(jnp.maximum(xt - oy + MARGIN, 0.0))

    @pl.when(pi == 0)
    def _init():
        out_ref[0, 0] = 0.0

    out_ref[0, 0] += s


_tc_hinge = pl.pallas_call(
    _tc_hinge_body,
    grid=(GRID,),
    in_specs=[
        pl.BlockSpec((C, BCOL), lambda i: (0, i)),
        pl.BlockSpec((1, 1, BCOL), lambda i: (i, 0, 0)),
    ],
    out_specs=pl.BlockSpec((1, 1), lambda i: (0, 0), memory_space=pltpu.SMEM),
    out_shape=jax.ShapeDtypeStruct((1, 1), jnp.float32),
)


def _sc_hinge_body(xt_hbm, y_hbm, out_hbm, ybuf, xbuf, accbuf, sem):
    wid = lax.axis_index("s") * NC + lax.axis_index("c")
    soff = pl.multiple_of(BT + wid * SW, SW)
    pltpu.sync_copy(y_hbm.at[pl.ds(soff, SW)], ybuf)
    pltpu.async_copy(xt_hbm.at[:, pl.ds(soff, SW)], xbuf, sem).wait()

    NA = 4

    accs = tuple(jnp.zeros((L,), jnp.float32) for _ in range(NA))
    for q in range(NG):
        yv = ybuf[pl.ds(q * L, L)]

        def s1(i, oyvs, yv=yv, q=q):
            c0 = i * UNROLL
            d = yv - c0
            oyvs = list(oyvs)
            for k in range(UNROLL):
                v = xbuf[c0 + k, pl.ds(q * L, L)]
                oyvs[k % NA] = jnp.where(d == k, v, oyvs[k % NA])
            return tuple(oyvs)

        oyvs = lax.fori_loop(0, C // UNROLL, s1,
                             tuple(jnp.zeros((L,), jnp.float32)
                                   for _ in range(NA)))
        ym = (oyvs[0] + oyvs[1]) + (oyvs[2] + oyvs[3]) - MARGIN

        def s2(i, accs_, ym=ym, q=q):
            c0 = i * UNROLL
            accs_ = list(accs_)
            for k in range(UNROLL):
                v = xbuf[c0 + k, pl.ds(q * L, L)]
                accs_[k % NA] = accs_[k % NA] + jnp.maximum(v - ym, 0.0)
            return tuple(accs_)

        accs = lax.fori_loop(0, C // UNROLL, s2, accs)

    accbuf[...] = (accs[0] + accs[1]) + (accs[2] + accs[3])
    pltpu.sync_copy(accbuf, out_hbm.at[pl.ds(wid * L, L)])


@functools.cache
def _sc_hinge():
    return pl.kernel(
        _sc_hinge_body,
        out_type=jax.ShapeDtypeStruct((NW * L,), jnp.float32),
        mesh=plsc.VectorSubcoreMesh(core_axis_name="c", subcore_axis_name="s",
                                    num_cores=NC, num_subcores=NS),
        scratch_types=[
            pltpu.VMEM((SW,), jnp.int32),
            pltpu.VMEM((C, SW), jnp.float32),
            pltpu.VMEM((L,), jnp.float32),
            pltpu.SemaphoreType.DMA,
        ],
    )


def kernel(output, y):
    y32 = y.astype(jnp.int32)
    xt = output.T
    sc_partials = _sc_hinge()(xt, y32)
    y3 = y32[:BT].reshape(GRID, 1, BCOL)
    tc_partial = _tc_hinge(xt, y3)
    total = tc_partial[0, 0] + jnp.sum(sc_partials)
    return (total - float(B)) / float(B)
